# baseline (speedup 1.0000x reference)
import functools

import jax
import jax.numpy as jnp
from jax import lax
from jax.experimental import pallas as pl
from jax.experimental.pallas import tpu as pltpu
from jax.experimental.pallas import tpu_sc as plsc

_N = 100000
_N2 = 100352
_E = 3200000
_NTILES = 32
_B = 128
_GRP = 4
_SBE = _B * _GRP
_SBT = 196
_E2 = _NTILES * _SBT * _SBE
_STRIPE = _N2 // 16
_ZR = 196


def _make_segsum(nc):
  mesh = plsc.VectorSubcoreMesh(core_axis_name="c", subcore_axis_name="s",
                                num_cores=2, num_subcores=16)

  @functools.partial(
      pl.kernel,
      out_type=jax.ShapeDtypeStruct((2, nc, _N2, 16), jnp.float32),
      mesh=mesh,
      scratch_types=[
          pltpu.VMEM((4, _SBE), jnp.int32),
          pltpu.VMEM((4, _GRP, _B), jnp.int32),
          pltpu.VMEM((4, _SBE), jnp.int32),
          pltpu.VMEM((2, _GRP, _B, 16), jnp.float32),
          pltpu.VMEM((_ZR, 16), jnp.float32),
          pltpu.VMEM_SHARED((_N2, 16), jnp.float32),
          pltpu.SemaphoreType.DMA,
          pltpu.SemaphoreType.DMA,
          pltpu.SemaphoreType.DMA,
          pltpu.SemaphoreType.DMA,
          pltpu.SemaphoreType.DMA,
          pltpu.SemaphoreType.DMA,
          pltpu.SemaphoreType.DMA,
          pltpu.SemaphoreType.DMA,
          pltpu.SemaphoreType.DMA,
      ],
      compiler_params=pltpu.CompilerParams(use_tc_tiling_on_sc=False),
  )
  def k(u2, src1, dst3, zrow, agg, sidx, didx, gidx, rows, zer, acc,
        semg0, semg1, semi0, semi1, semi2, semi3, semz, sems0, sems1):
    semg = [semg0, semg1]
    semi = [semi0, semi1, semi2, semi3]
    sems = [sems0, sems1]
    cid = lax.axis_index("c")
    sid = lax.axis_index("s")
    tile = cid * 16 + sid
    stripe = sid * _STRIPE
    sb_base = tile * _SBT
    pltpu.sync_copy(zrow, zer)

    def idx_fire(sb, st):
      gsb = sb_base + jnp.minimum(sb, _SBT - 1)
      pltpu.async_copy(src1.at[pl.ds(gsb * _SBE, _SBE)], sidx.at[st],
                       semi[st])
      pltpu.async_copy(dst3.at[pl.ds(gsb * _GRP, _GRP)], didx.at[st],
                       semi[st])

    def idx_wait(st):
      pltpu.make_async_copy(src1.at[pl.ds(0, _SBE)], sidx.at[st],
                            semi[st]).wait()
      pltpu.make_async_copy(dst3.at[pl.ds(0, _GRP)], didx.at[st],
                            semi[st]).wait()

    def gidx_fill(st, f):
      def gix(j, _):
        for t in range(4):
          off = j * 64 + t * 16
          gidx[st, pl.ds(off, 16)] = sidx[st, pl.ds(off, 16)] * nc + f
        return 0
      lax.fori_loop(0, _SBE // 64, gix, 0)

    def gath_fire(q, st):
      for b in range(_GRP):
        pltpu.async_copy(u2.at[gidx.at[st, pl.ds(b * _B, _B)]],
                         rows.at[q, b], semg[q])

    def drain_fire(q, st):
      for b in range(_GRP):
        pltpu.make_async_copy(u2.at[pl.ds(0, _B)], rows.at[q, b],
                              semg[q]).wait()
        pltpu.async_copy(rows.at[q, b], acc.at[didx.at[st, b]], sems[q],
                         add=True)

    def gath_drain(q):
      for b in range(_GRP):
        pltpu.make_async_copy(u2.at[pl.ds(0, _B)], rows.at[q, b],
                              semg[q]).wait()

    def scat_wait(q):
      for b in range(_GRP):
        pltpu.make_async_copy(rows.at[q, b], acc.at[pl.ds(0, _B)],
                              sems[q]).wait()

    def chunk_body(f, _):
      def zf(i, _):
        pltpu.async_copy(zer, acc.at[pl.ds(stripe + i * _ZR, _ZR)], semz)
        return 0
      lax.fori_loop(0, _STRIPE // _ZR, zf, 0)

      def zw(i, _):
        pltpu.make_async_copy(zer, acc.at[pl.ds(stripe, _ZR)], semz).wait()
        return 0
      lax.fori_loop(0, _STRIPE // _ZR, zw, 0)
      plsc.subcore_barrier()

      for st in range(4):
        idx_fire(st, st)
      idx_wait(0)
      gidx_fill(0, f)
      gath_fire(0, 0)

      def quad_body(i, _):
        idx_wait(1)
        gidx_fill(1, f)
        drain_fire(0, 0)

        @pl.when(i > 0)
        def _():
          scat_wait(1)
        gath_fire(1, 1)
        idx_fire(4 * i + 4, 0)
        idx_wait(2)
        gidx_fill(2, f)
        drain_fire(1, 1)
        scat_wait(0)
        gath_fire(0, 2)
        idx_fire(4 * i + 5, 1)
        idx_wait(3)
        gidx_fill(3, f)
        drain_fire(0, 2)
        scat_wait(1)
        gath_fire(1, 3)
        idx_fire(4 * i + 6, 2)
        idx_wait(0)
        gidx_fill(0, f)
        drain_fire(1, 3)
        scat_wait(0)
        gath_fire(0, 0)
        idx_fire(4 * i + 7, 3)
        return 0
      lax.fori_loop(0, _SBT // 4, quad_body, 0)
      gath_drain(0)
      scat_wait(1)
      for st in (1, 2, 3):
        idx_wait(st)
      plsc.subcore_barrier()
      pltpu.sync_copy(acc.at[pl.ds(stripe, _STRIPE)],
                      agg.at[cid, f, pl.ds(stripe, _STRIPE)])
      return 0
    lax.fori_loop(0, nc, chunk_body, 0)

  return k


def _segsum(u, src1, dst3, zrow):
  d = u.shape[1]
  nc = d // 16
  return _make_segsum(nc)(u.reshape(_N2 * nc, 16), src1, dst3, zrow)


_R = 896


def _tc_call(body, n, in_specs_widths, out_widths):
  grid = (n // _R,)
  in_specs = []
  for kind, w in in_specs_widths:
    if kind == "rows":
      in_specs.append(pl.BlockSpec((_R, w), lambda i: (i, 0)))
    elif kind == "agg":
      in_specs.append(
          pl.BlockSpec((2, w // 16, _R, 16), lambda i: (0, 0, i, 0)))
    else:
      in_specs.append(
          pl.BlockSpec(kind, lambda i, _r=len(kind): (0,) * _r))
  out_shapes = tuple(jax.ShapeDtypeStruct((n, w), jnp.float32)
                     for w in out_widths)
  out_specs = tuple(pl.BlockSpec((_R, w), lambda i: (i, 0))
                    for w in out_widths)
  if len(out_widths) == 1:
    out_shapes, out_specs = out_shapes[0], out_specs[0]
  return pl.pallas_call(body, grid=grid, in_specs=in_specs,
                        out_shape=out_shapes, out_specs=out_specs)


def _agg_rows(a):
  s = a[0] + a[1]
  nc = s.shape[0]
  if nc == 1:
    return s[0]
  return jnp.concatenate([s[fc] for fc in range(nc)], axis=-1)


def _mlp_conv(h, agg, w1, b1, w2, b2, outer_relu):
  din, d1 = w1.shape
  d2 = w2.shape[1]

  def body(h_ref, a_ref, w1r, b1r, w2r, b2r, o_ref):
    z = h_ref[...] + _agg_rows(a_ref[...])
    z = jnp.dot(z, w1r[...], preferred_element_type=jnp.float32) + b1r[...]
    z = jnp.maximum(z, 0.0)
    z = jnp.dot(z, w2r[...], preferred_element_type=jnp.float32) + b2r[...]
    if outer_relu:
      z = jnp.maximum(z, 0.0)
    o_ref[...] = z

  return _tc_call(
      body, h.shape[0],
      [("rows", din), ("agg", din), ((din, d1), None), ((1, d1), None),
       ((d1, d2), None), ((1, d2), None)],
      (d2,),
  )(h, agg, w1, b1.reshape(1, -1), w2, b2.reshape(1, -1))


def _mlp_conv_premul(h, agg, w1, b1, w2, b2, w3):
  din, d1 = w1.shape
  d2 = w2.shape[1]
  d3 = w3.shape[1]

  def body(h_ref, a_ref, w1r, b1r, w2r, b2r, w3r, o_ref):
    z = h_ref[...] + _agg_rows(a_ref[...])
    z = jnp.dot(z, w1r[...], preferred_element_type=jnp.float32) + b1r[...]
    z = jnp.maximum(z, 0.0)
    z = jnp.dot(z, w2r[...], preferred_element_type=jnp.float32) + b2r[...]
    z = jnp.maximum(z, 0.0)
    o_ref[...] = jnp.dot(z, w3r[...], preferred_element_type=jnp.float32)

  return _tc_call(
      body, h.shape[0],
      [("rows", din), ("agg", din), ((din, d1), None), ((1, d1), None),
       ((d1, d2), None), ((1, d2), None), ((d2, d3), None)],
      (d3,),
  )(h, agg, w1, b1.reshape(1, -1), w2, b2.reshape(1, -1), w3)


def _head_call(u3, agg3, b31, w32, b32, wm, bm, w41):
  n = u3.shape[0]
  wm8 = jnp.zeros((64, 8), jnp.float32).at[:, 0:1].set(wm)
  bm8 = jnp.zeros((1, 8), jnp.float32).at[0, 0].set(bm[0])
  w41p = jnp.zeros((64, 16), jnp.float32).at[:, 0:1].set(w41)

  def body(u_ref, a_ref, b31r, w32r, b32r, wmr, bmr, w41r, u4_ref, mm_ref):
    z = u_ref[...] + _agg_rows(a_ref[...]) + b31r[...]
    z = jnp.maximum(z, 0.0)
    lat = jnp.dot(z, w32r[...], preferred_element_type=jnp.float32) + b32r[...]
    mm = jnp.dot(lat, wmr[...], preferred_element_type=jnp.float32) + bmr[...]
    mm_ref[...] = jnp.where(mm >= 0.0, mm, 0.01 * mm)
    u4_ref[...] = jnp.dot(lat, w41r[...], preferred_element_type=jnp.float32)

  return _tc_call(
      body, n,
      [("rows", 64), ("agg", 64), ((1, 64), None), ((64, 64), None),
       ((1, 64), None), ((64, 8), None), ((1, 8), None), ((64, 16), None)],
      (16, 8),
  )(u3, agg3, b31.reshape(1, -1), w32, b32.reshape(1, -1), wm8, bm8, w41p)


def _d_call(u4, agg4, b41, w42, b42):
  n = u4.shape[0]
  b41p = jnp.zeros((1, 16), jnp.float32).at[0, 0].set(b41[0])
  sc = jnp.full((1, 16), w42[0, 0], jnp.float32)
  off = jnp.full((1, 16), b42[0], jnp.float32)

  def body(u_ref, a_ref, br, scr, offr, o_ref):
    z = u_ref[...] + _agg_rows(a_ref[...]) + br[...]
    z = jnp.maximum(z, 0.0)
    o_ref[...] = z * scr[...] + offr[...]

  return _tc_call(
      body, n,
      [("rows", 16), ("agg", 16), ((1, 16), None), ((1, 16), None),
       ((1, 16), None)],
      (16,),
  )(u4, agg4, b41p, sc, off)


def kernel(x, edge_index, params):
  n = x.shape[0]
  pad_e = _E2 - _E
  src1 = jnp.concatenate(
      [edge_index[0], jnp.zeros((pad_e,), jnp.int32)])
  dst3 = jnp.concatenate(
      [edge_index[1], jnp.full((pad_e,), n, jnp.int32)]).reshape(
          _E2 // _B, _B)
  zrow = jnp.zeros((_ZR, 16), jnp.float32)

  g = params["gin1"]
  xpad = jnp.pad(x, ((0, _N2 - n), (0, 32 - x.shape[1])))
  w1p = jnp.pad(g[0][0]["W"], ((0, 32 - x.shape[1]), (0, 0)))

  agg0 = _segsum(xpad, src1, dst3, zrow)
  h1 = _mlp_conv(xpad, agg0, w1p, g[0][0]["b"], g[0][1]["W"], g[0][1]["b"],
                 outer_relu=True)
  agg1 = _segsum(h1, src1, dst3, zrow)
  h2 = _mlp_conv(h1, agg1, g[1][0]["W"], g[1][0]["b"], g[1][1]["W"],
                 g[1][1]["b"], outer_relu=True)
  agg2 = _segsum(h2, src1, dst3, zrow)
  u3 = _mlp_conv_premul(h2, agg2, g[2][0]["W"], g[2][0]["b"], g[2][1]["W"],
                        g[2][1]["b"], g[3][0]["W"])
  agg3 = _segsum(u3, src1, dst3, zrow)
  u4, mmse8 = _head_call(u3, agg3, g[3][0]["b"], g[3][1]["W"], g[3][1]["b"],
                         params["mmse"]["W"], params["mmse"]["b"],
                         params["gin2"][0][0]["W"])
  agg4 = _segsum(u4, src1, dst3, zrow)
  d16 = _d_call(u4, agg4, params["gin2"][0][0]["b"],
                params["gin2"][0][1]["W"], params["gin2"][0][1]["b"])
  return d16[:n, :1], mmse8[:n, :1]


# speedup vs baseline: 1.2670x; 1.2670x over previous
import functools

import jax
import jax.numpy as jnp
from jax import lax
from jax.experimental import pallas as pl
from jax.experimental.pallas import tpu as pltpu
from jax.experimental.pallas import tpu_sc as plsc

_N = 100000
_N2 = 100352
_E = 3200000
_NTILES = 32
_B = 128
_GRP = 4
_SBE = _B * _GRP
_SBT = 196
_E2 = _NTILES * _SBT * _SBE
_STRIPE = _N2 // 16
_ZR = 196


def _make_segsum(nc):
  mesh = plsc.VectorSubcoreMesh(core_axis_name="c", subcore_axis_name="s",
                                num_cores=2, num_subcores=16)

  @functools.partial(
      pl.kernel,
      out_type=jax.ShapeDtypeStruct((2, nc, _N2, 16), jnp.float32),
      mesh=mesh,
      scratch_types=[
          pltpu.VMEM((4, _SBE), jnp.int32),
          pltpu.VMEM((4, _GRP, _B), jnp.int32),
          pltpu.VMEM((4, _SBE), jnp.int32),
          pltpu.VMEM((2, _GRP, _B, 16), jnp.float32),
          pltpu.VMEM((_ZR, 16), jnp.float32),
          pltpu.VMEM_SHARED((_N2, 16), jnp.float32),
          pltpu.SemaphoreType.DMA,
          pltpu.SemaphoreType.DMA,
          pltpu.SemaphoreType.DMA,
          pltpu.SemaphoreType.DMA,
          pltpu.SemaphoreType.DMA,
          pltpu.SemaphoreType.DMA,
          pltpu.SemaphoreType.DMA,
          pltpu.SemaphoreType.DMA,
          pltpu.SemaphoreType.DMA,
      ],
      compiler_params=pltpu.CompilerParams(use_tc_tiling_on_sc=False),
  )
  def k(u2, src1, dst3, zrow, agg, sidx, didx, gidx, rows, zer, acc,
        semg0, semg1, semi0, semi1, semi2, semi3, semz, sems0, sems1):
    semg = [semg0, semg1]
    semi = [semi0, semi1, semi2, semi3]
    sems = [sems0, sems1]
    cid = lax.axis_index("c")
    sid = lax.axis_index("s")
    tile = cid * 16 + sid
    stripe = sid * _STRIPE
    sb_base = tile * _SBT
    pltpu.sync_copy(zrow, zer)

    def idx_fire(sb, st):
      gsb = sb_base + jnp.minimum(sb, _SBT - 1)
      pltpu.async_copy(src1.at[pl.ds(gsb * _SBE, _SBE)], sidx.at[st],
                       semi[st])
      pltpu.async_copy(dst3.at[pl.ds(gsb * _GRP, _GRP)], didx.at[st],
                       semi[st])

    def idx_wait(st):
      pltpu.make_async_copy(src1.at[pl.ds(0, _SBE)], sidx.at[st],
                            semi[st]).wait()
      pltpu.make_async_copy(dst3.at[pl.ds(0, _GRP)], didx.at[st],
                            semi[st]).wait()

    def gidx_fill(st, f):
      def gix(j, _):
        for t in range(4):
          off = j * 64 + t * 16
          gidx[st, pl.ds(off, 16)] = sidx[st, pl.ds(off, 16)] * nc + f
        return 0
      lax.fori_loop(0, _SBE // 64, gix, 0)

    def gath_fire(q, st):
      for b in range(_GRP):
        pltpu.async_copy(u2.at[gidx.at[st, pl.ds(b * _B, _B)]],
                         rows.at[q, b], semg[q])

    def drain_fire(q, st):
      for b in range(_GRP):
        pltpu.make_async_copy(u2.at[pl.ds(0, _B)], rows.at[q, b],
                              semg[q]).wait()
        pltpu.async_copy(rows.at[q, b], acc.at[didx.at[st, b]], sems[q],
                         add=True)

    def gath_drain(q):
      for b in range(_GRP):
        pltpu.make_async_copy(u2.at[pl.ds(0, _B)], rows.at[q, b],
                              semg[q]).wait()

    def scat_wait(q):
      for b in range(_GRP):
        pltpu.make_async_copy(rows.at[q, b], acc.at[pl.ds(0, _B)],
                              sems[q]).wait()

    def chunk_body(f, _):
      def zf(i, _):
        pltpu.async_copy(zer, acc.at[pl.ds(stripe + i * _ZR, _ZR)], semz)
        return 0
      lax.fori_loop(0, _STRIPE // _ZR, zf, 0)

      def zw(i, _):
        pltpu.make_async_copy(zer, acc.at[pl.ds(stripe, _ZR)], semz).wait()
        return 0
      lax.fori_loop(0, _STRIPE // _ZR, zw, 0)
      plsc.subcore_barrier()

      for st in range(4):
        idx_fire(st, st)
      idx_wait(0)
      gidx_fill(0, f)
      gath_fire(0, 0)

      def quad_body(i, _):
        idx_wait(1)
        gidx_fill(1, f)

        @pl.when(i > 0)
        def _():
          scat_wait(1)
        gath_fire(1, 1)
        drain_fire(0, 0)
        idx_fire(4 * i + 4, 0)
        idx_wait(2)
        gidx_fill(2, f)
        scat_wait(0)
        gath_fire(0, 2)
        drain_fire(1, 1)
        idx_fire(4 * i + 5, 1)
        idx_wait(3)
        gidx_fill(3, f)
        scat_wait(1)
        gath_fire(1, 3)
        drain_fire(0, 2)
        idx_fire(4 * i + 6, 2)
        idx_wait(0)
        gidx_fill(0, f)
        scat_wait(0)
        gath_fire(0, 0)
        drain_fire(1, 3)
        idx_fire(4 * i + 7, 3)
        return 0
      lax.fori_loop(0, _SBT // 4, quad_body, 0)
      gath_drain(0)
      scat_wait(1)
      for st in (1, 2, 3):
        idx_wait(st)
      plsc.subcore_barrier()
      pltpu.sync_copy(acc.at[pl.ds(stripe, _STRIPE)],
                      agg.at[cid, f, pl.ds(stripe, _STRIPE)])
      return 0
    lax.fori_loop(0, nc, chunk_body, 0)

  return k


def _segsum(u, src1, dst3, zrow):
  d = u.shape[1]
  nc = d // 16
  return _make_segsum(nc)(u.reshape(_N2 * nc, 16), src1, dst3, zrow)


_R = 896


def _tc_call(body, n, in_specs_widths, out_widths):
  grid = (n // _R,)
  in_specs = []
  for kind, w in in_specs_widths:
    if kind == "rows":
      in_specs.append(pl.BlockSpec((_R, w), lambda i: (i, 0)))
    elif kind == "agg":
      in_specs.append(
          pl.BlockSpec((2, w // 16, _R, 16), lambda i: (0, 0, i, 0)))
    else:
      in_specs.append(
          pl.BlockSpec(kind, lambda i, _r=len(kind): (0,) * _r))
  out_shapes = tuple(jax.ShapeDtypeStruct((n, w), jnp.float32)
                     for w in out_widths)
  out_specs = tuple(pl.BlockSpec((_R, w), lambda i: (i, 0))
                    for w in out_widths)
  if len(out_widths) == 1:
    out_shapes, out_specs = out_shapes[0], out_specs[0]
  return pl.pallas_call(body, grid=grid, in_specs=in_specs,
                        out_shape=out_shapes, out_specs=out_specs)


def _agg_rows(a):
  s = a[0] + a[1]
  nc = s.shape[0]
  if nc == 1:
    return s[0]
  return jnp.concatenate([s[fc] for fc in range(nc)], axis=-1)


def _mlp_conv(h, agg, w1, b1, w2, b2, outer_relu):
  din, d1 = w1.shape
  d2 = w2.shape[1]

  def body(h_ref, a_ref, w1r, b1r, w2r, b2r, o_ref):
    z = h_ref[...] + _agg_rows(a_ref[...])
    z = jnp.dot(z, w1r[...], preferred_element_type=jnp.float32) + b1r[...]
    z = jnp.maximum(z, 0.0)
    z = jnp.dot(z, w2r[...], preferred_element_type=jnp.float32) + b2r[...]
    if outer_relu:
      z = jnp.maximum(z, 0.0)
    o_ref[...] = z

  return _tc_call(
      body, h.shape[0],
      [("rows", din), ("agg", din), ((din, d1), None), ((1, d1), None),
       ((d1, d2), None), ((1, d2), None)],
      (d2,),
  )(h, agg, w1, b1.reshape(1, -1), w2, b2.reshape(1, -1))


def _mlp_conv_premul(h, agg, w1, b1, w2, b2, w3):
  din, d1 = w1.shape
  d2 = w2.shape[1]
  d3 = w3.shape[1]

  def body(h_ref, a_ref, w1r, b1r, w2r, b2r, w3r, o_ref):
    z = h_ref[...] + _agg_rows(a_ref[...])
    z = jnp.dot(z, w1r[...], preferred_element_type=jnp.float32) + b1r[...]
    z = jnp.maximum(z, 0.0)
    z = jnp.dot(z, w2r[...], preferred_element_type=jnp.float32) + b2r[...]
    z = jnp.maximum(z, 0.0)
    o_ref[...] = jnp.dot(z, w3r[...], preferred_element_type=jnp.float32)

  return _tc_call(
      body, h.shape[0],
      [("rows", din), ("agg", din), ((din, d1), None), ((1, d1), None),
       ((d1, d2), None), ((1, d2), None), ((d2, d3), None)],
      (d3,),
  )(h, agg, w1, b1.reshape(1, -1), w2, b2.reshape(1, -1), w3)


def _head_call(u3, agg3, b31, w32, b32, wm, bm, w41):
  n = u3.shape[0]
  wm8 = jnp.zeros((64, 8), jnp.float32).at[:, 0:1].set(wm)
  bm8 = jnp.zeros((1, 8), jnp.float32).at[0, 0].set(bm[0])
  w41p = jnp.zeros((64, 16), jnp.float32).at[:, 0:1].set(w41)

  def body(u_ref, a_ref, b31r, w32r, b32r, wmr, bmr, w41r, u4_ref, mm_ref):
    z = u_ref[...] + _agg_rows(a_ref[...]) + b31r[...]
    z = jnp.maximum(z, 0.0)
    lat = jnp.dot(z, w32r[...], preferred_element_type=jnp.float32) + b32r[...]
    mm = jnp.dot(lat, wmr[...], preferred_element_type=jnp.float32) + bmr[...]
    mm_ref[...] = jnp.where(mm >= 0.0, mm, 0.01 * mm)
    u4_ref[...] = jnp.dot(lat, w41r[...], preferred_element_type=jnp.float32)

  return _tc_call(
      body, n,
      [("rows", 64), ("agg", 64), ((1, 64), None), ((64, 64), None),
       ((1, 64), None), ((64, 8), None), ((1, 8), None), ((64, 16), None)],
      (16, 8),
  )(u3, agg3, b31.reshape(1, -1), w32, b32.reshape(1, -1), wm8, bm8, w41p)


def _d_call(u4, agg4, b41, w42, b42):
  n = u4.shape[0]
  b41p = jnp.zeros((1, 16), jnp.float32).at[0, 0].set(b41[0])
  sc = jnp.full((1, 16), w42[0, 0], jnp.float32)
  off = jnp.full((1, 16), b42[0], jnp.float32)

  def body(u_ref, a_ref, br, scr, offr, o_ref):
    z = u_ref[...] + _agg_rows(a_ref[...]) + br[...]
    z = jnp.maximum(z, 0.0)
    o_ref[...] = z * scr[...] + offr[...]

  return _tc_call(
      body, n,
      [("rows", 16), ("agg", 16), ((1, 16), None), ((1, 16), None),
       ((1, 16), None)],
      (16,),
  )(u4, agg4, b41p, sc, off)


def kernel(x, edge_index, params):
  n = x.shape[0]
  pad_e = _E2 - _E
  src1 = jnp.concatenate(
      [edge_index[0], jnp.zeros((pad_e,), jnp.int32)])
  dst3 = jnp.concatenate(
      [edge_index[1], jnp.full((pad_e,), n, jnp.int32)]).reshape(
          _E2 // _B, _B)
  zrow = jnp.zeros((_ZR, 16), jnp.float32)

  g = params["gin1"]
  xpad = jnp.pad(x, ((0, _N2 - n), (0, 32 - x.shape[1])))
  w1p = jnp.pad(g[0][0]["W"], ((0, 32 - x.shape[1]), (0, 0)))

  agg0 = _segsum(xpad, src1, dst3, zrow)
  h1 = _mlp_conv(xpad, agg0, w1p, g[0][0]["b"], g[0][1]["W"], g[0][1]["b"],
                 outer_relu=True)
  agg1 = _segsum(h1, src1, dst3, zrow)
  h2 = _mlp_conv(h1, agg1, g[1][0]["W"], g[1][0]["b"], g[1][1]["W"],
                 g[1][1]["b"], outer_relu=True)
  agg2 = _segsum(h2, src1, dst3, zrow)
  u3 = _mlp_conv_premul(h2, agg2, g[2][0]["W"], g[2][0]["b"], g[2][1]["W"],
                        g[2][1]["b"], g[3][0]["W"])
  agg3 = _segsum(u3, src1, dst3, zrow)
  u4, mmse8 = _head_call(u3, agg3, g[3][0]["b"], g[3][1]["W"], g[3][1]["b"],
                         params["mmse"]["W"], params["mmse"]["b"],
                         params["gin2"][0][0]["W"])
  agg4 = _segsum(u4, src1, dst3, zrow)
  d16 = _d_call(u4, agg4, params["gin2"][0][0]["b"],
                params["gin2"][0][1]["W"], params["gin2"][0][1]["b"])
  return d16[:n, :1], mmse8[:n, :1]
